# baseline (speedup 1.0000x reference)
import functools

import jax
import jax.numpy as jnp
from jax import lax
from jax.experimental import pallas as pl
from jax.experimental.pallas import tpu as pltpu
from jax.experimental.pallas import tpu_sc as plsc

_NC = 2
_NS = 16


def _sc_gather(word_table, token_indices):
    V, D = word_table.shape
    (B,) = token_indices.shape
    NW = _NC * _NS
    b_per_w = B // NW
    C = 32
    n_chunks = b_per_w // C

    mesh = plsc.VectorSubcoreMesh(core_axis_name="c", subcore_axis_name="s")

    @functools.partial(
        pl.kernel,
        mesh=mesh,
        out_type=jax.ShapeDtypeStruct((B, D), jnp.float32),
        scratch_types=[
            pltpu.VMEM((b_per_w,), jnp.int32),
            pltpu.VMEM((C, D), jnp.float32),
            pltpu.VMEM((C, D), jnp.float32),
            pltpu.SemaphoreType.DMA,
            pltpu.SemaphoreType.DMA,
            pltpu.SemaphoreType.DMA,
            pltpu.SemaphoreType.DMA,
        ],
    )
    def gather_kernel(table_hbm, idx_hbm, out_hbm, idx_v, rows0, rows1,
                      g0, g1, s0, s1):
        wid = lax.axis_index("s") * _NC + lax.axis_index("c")
        base = wid * b_per_w
        pltpu.sync_copy(idx_hbm.at[pl.ds(base, b_per_w)], idx_v)

        bufs = (rows0, rows1)
        gsems = (g0, g1)
        ssems = (s0, s1)

        def start_gather(i):
            pltpu.make_async_copy(
                table_hbm.at[idx_v.at[pl.ds(i * C, C)]], bufs[i % 2],
                gsems[i % 2]).start()

        def start_out(i):
            pltpu.make_async_copy(
                bufs[i % 2], out_hbm.at[pl.ds(base + i * C, C)],
                ssems[i % 2]).start()

        def wait_gather(i):
            pltpu.make_async_copy(
                table_hbm.at[idx_v.at[pl.ds(i * C, C)]], bufs[i % 2],
                gsems[i % 2]).wait()

        def wait_out(i):
            pltpu.make_async_copy(
                bufs[i % 2], out_hbm.at[pl.ds(base + i * C, C)],
                ssems[i % 2]).wait()

        start_gather(0)
        for i in range(n_chunks):
            if i + 1 < n_chunks:
                if i >= 1:
                    wait_out(i - 1)
                start_gather(i + 1)
            wait_gather(i)
            start_out(i)
        wait_out(n_chunks - 2)
        wait_out(n_chunks - 1)

    return gather_kernel(word_table, token_indices)


def _tc_matmul(gathered, tag_indices, W1, tag_table, b1):
    N, D = gathered.shape
    CD, DT = W1.shape
    TAGS, TD = tag_table.shape
    TILE = 2048
    grid = (N // TILE,)

    tag3 = tag_indices.astype(jnp.int32).reshape(N // TILE, 1, TILE)
    b2 = b1.reshape(1, CD)

    def body(tok_ref, tag_ref, w1_ref, tt_ref, b_ref, out_ref):
        tok = tok_ref[...].astype(jnp.bfloat16)
        w1w = w1_ref[:, :D].astype(jnp.bfloat16)
        w1t = w1_ref[:, D:]
        t = lax.dot_general(tt_ref[...], w1t, (((1,), (1,)), ((), ())),
                            preferred_element_type=jnp.float32)
        tags = tag_ref[0, 0, :]
        oh = (tags[:, None]
              == lax.broadcasted_iota(jnp.int32, (TILE, TAGS), 1)
              ).astype(jnp.bfloat16)
        acc = lax.dot_general(tok, w1w, (((1,), (1,)), ((), ())),
                              preferred_element_type=jnp.float32)
        acc = acc + lax.dot_general(oh, t.astype(jnp.bfloat16),
                                    (((1,), (0,)), ((), ())),
                                    preferred_element_type=jnp.float32)
        out_ref[...] = acc + b_ref[...]

    return pl.pallas_call(
        body,
        grid=grid,
        in_specs=[
            pl.BlockSpec((TILE, D), lambda i: (i, 0)),
            pl.BlockSpec((1, 1, TILE), lambda i: (i, 0, 0)),
            pl.BlockSpec((CD, DT), lambda i: (0, 0)),
            pl.BlockSpec((TAGS, TD), lambda i: (0, 0)),
            pl.BlockSpec((1, CD), lambda i: (0, 0)),
        ],
        out_specs=pl.BlockSpec((TILE, CD), lambda i: (i, 0)),
        out_shape=jax.ShapeDtypeStruct((N, CD), jnp.float32),
    )(gathered, tag3, W1, tag_table, b2)


def kernel(token_indices, tag_indices, word_table, tag_table, W1, b1):
    tok = token_indices.astype(jnp.int32)
    (N,) = tok.shape
    K = 4
    chunk = N // K
    gathered = [
        _sc_gather(word_table, lax.slice(tok, (k * chunk,), ((k + 1) * chunk,)))
        for k in range(K)
    ]
    outs = [
        _tc_matmul(gathered[k],
                   lax.slice(tag_indices, (k * chunk,), ((k + 1) * chunk,)),
                   W1, tag_table, b1)
        for k in range(K)
    ]
    return jnp.concatenate(outs, axis=0)


# speedup vs baseline: 1.4363x; 1.4363x over previous
import functools

import jax
import jax.numpy as jnp
from jax import lax
from jax.experimental import pallas as pl
from jax.experimental.pallas import tpu as pltpu
from jax.experimental.pallas import tpu_sc as plsc

_NC = 2
_NS = 16


def _sc_gather(word_table, token_indices):
    V, D = word_table.shape
    (B,) = token_indices.shape
    NW = _NC * _NS
    b_per_w = B // NW
    C = 32
    n_chunks = b_per_w // C

    mesh = plsc.VectorSubcoreMesh(core_axis_name="c", subcore_axis_name="s")

    @functools.partial(
        pl.kernel,
        mesh=mesh,
        out_type=jax.ShapeDtypeStruct((B, D), jnp.float32),
        scratch_types=[
            pltpu.VMEM((b_per_w,), jnp.int32),
            pltpu.VMEM((C, D), jnp.float32),
            pltpu.VMEM((C, D), jnp.float32),
            pltpu.SemaphoreType.DMA,
            pltpu.SemaphoreType.DMA,
            pltpu.SemaphoreType.DMA,
            pltpu.SemaphoreType.DMA,
        ],
    )
    def gather_kernel(table_hbm, idx_hbm, out_hbm, idx_v, rows0, rows1,
                      g0, g1, s0, s1):
        wid = lax.axis_index("s") * _NC + lax.axis_index("c")
        base = wid * b_per_w
        pltpu.sync_copy(idx_hbm.at[pl.ds(base, b_per_w)], idx_v)

        bufs = (rows0, rows1)
        gsems = (g0, g1)
        ssems = (s0, s1)

        def start_gather(i):
            pltpu.make_async_copy(
                table_hbm.at[idx_v.at[pl.ds(i * C, C)]], bufs[i % 2],
                gsems[i % 2]).start()

        def start_out(i):
            pltpu.make_async_copy(
                bufs[i % 2], out_hbm.at[pl.ds(base + i * C, C)],
                ssems[i % 2]).start()

        def wait_gather(i):
            pltpu.make_async_copy(
                table_hbm.at[idx_v.at[pl.ds(i * C, C)]], bufs[i % 2],
                gsems[i % 2]).wait()

        def wait_out(i):
            pltpu.make_async_copy(
                bufs[i % 2], out_hbm.at[pl.ds(base + i * C, C)],
                ssems[i % 2]).wait()

        start_gather(0)
        for i in range(n_chunks):
            if i + 1 < n_chunks:
                if i >= 1:
                    wait_out(i - 1)
                start_gather(i + 1)
            wait_gather(i)
            start_out(i)
        wait_out(n_chunks - 2)
        wait_out(n_chunks - 1)

    return gather_kernel(word_table, token_indices)


def _tc_matmul_chunk(prev, gathered_k, tag3_k, W1bf, ttbf, b2, k, N, TILE):
    chunk, D = gathered_k.shape
    CD, DT = W1bf.shape
    TAGS, TD = ttbf.shape
    tiles = chunk // TILE

    def body(*refs):
        if prev is None:
            tok_ref, tag_ref, w1_ref, tt_ref, b_ref, out_ref = refs
        else:
            _, tok_ref, tag_ref, w1_ref, tt_ref, b_ref, out_ref = refs
        tok = tok_ref[...].astype(jnp.bfloat16)
        w1w = w1_ref[:, :D]
        w1t = w1_ref[:, D:]
        t = lax.dot_general(tt_ref[...], w1t, (((1,), (1,)), ((), ())),
                            preferred_element_type=jnp.float32)
        tags = tag_ref[0, 0, :]
        oh = (tags[:, None]
              == lax.broadcasted_iota(jnp.int32, (TILE, TAGS), 1)
              ).astype(jnp.bfloat16)
        acc = lax.dot_general(tok, w1w, (((1,), (1,)), ((), ())),
                              preferred_element_type=jnp.float32)
        acc = acc + lax.dot_general(oh, t.astype(jnp.bfloat16),
                                    (((1,), (0,)), ((), ())),
                                    preferred_element_type=jnp.float32)
        out_ref[...] = acc + b_ref[...]

    in_specs = [
        pl.BlockSpec((TILE, D), lambda i: (i, 0)),
        pl.BlockSpec((1, 1, TILE), lambda i: (i, 0, 0)),
        pl.BlockSpec((CD, DT), lambda i: (0, 0)),
        pl.BlockSpec((TAGS, TD), lambda i: (0, 0)),
        pl.BlockSpec((1, CD), lambda i: (0, 0)),
    ]
    args = [gathered_k, tag3_k, W1bf, ttbf, b2]
    aliases = {}
    if prev is not None:
        in_specs = [pl.BlockSpec(memory_space=pl.ANY)] + in_specs
        args = [prev] + args
        aliases = {0: 0}

    return pl.pallas_call(
        body,
        grid=(tiles,),
        in_specs=in_specs,
        out_specs=pl.BlockSpec((TILE, CD), lambda i: (k * tiles + i, 0)),
        out_shape=jax.ShapeDtypeStruct((N, CD), jnp.float32),
        input_output_aliases=aliases,
    )(*args)


def kernel(token_indices, tag_indices, word_table, tag_table, W1, b1):
    tok = token_indices.astype(jnp.int32)
    tags = tag_indices.astype(jnp.int32)
    (N,) = tok.shape
    CD = W1.shape[0]
    K = 4
    TILE = 2048
    chunk = N // K

    W1bf = W1.astype(jnp.bfloat16)
    ttbf = tag_table.astype(jnp.bfloat16)
    b2 = b1.reshape(1, CD)

    gathered = [
        _sc_gather(word_table, lax.slice(tok, (k * chunk,), ((k + 1) * chunk,)))
        for k in range(K)
    ]
    out = None
    for k in range(K):
        tag3_k = lax.slice(tags, (k * chunk,), ((k + 1) * chunk,)).reshape(
            chunk // TILE, 1, TILE)
        out = _tc_matmul_chunk(out, gathered[k], tag3_k, W1bf, ttbf, b2,
                               k, N, TILE)
    return out
